# Initial kernel scaffold; baseline (speedup 1.0000x reference)
import jax
import jax.numpy as jnp
from jax.experimental import pallas as pl


def kernel(input, target):
    raise NotImplementedError("write your pallas kernel here")


# speedup vs baseline: 12.6991x; 12.6991x over previous
import functools

import jax
import jax.numpy as jnp
from jax.experimental import pallas as pl
from jax.experimental.pallas import tpu as pltpu

_THRESHOLD = 0.1
_SUB = 8


def _top2_count_kernel(x_ref, t_ref, o_ref, *, num_classes):
    b = pl.program_id(0)
    tv = t_ref[0]
    shape = tv.shape
    m1 = jnp.full(shape, -jnp.inf, jnp.float32)
    m2 = jnp.full(shape, -jnp.inf, jnp.float32)
    i1 = jnp.zeros(shape, jnp.int32)
    i2 = jnp.zeros(shape, jnp.int32)
    for k in range(num_classes):
        v = x_ref[0, k]
        gt1 = v > m1
        gt2 = v > m2
        m2 = jnp.where(gt1, m1, jnp.where(gt2, v, m2))
        i2 = jnp.where(gt1, i1, jnp.where(gt2, k, i2))
        m1 = jnp.where(gt1, v, m1)
        i1 = jnp.where(gt1, k, i1)
    c1 = (i1 == tv).astype(jnp.int32)
    c2 = jnp.logical_and(m1 - m2 < _THRESHOLD, i2 == tv).astype(jnp.int32)
    cnt = jnp.sum(c1) + jnp.sum(c2)

    @pl.when(b == 0)
    def _init():
        o_ref[0, 0] = 0

    o_ref[0, 0] += cnt


def kernel(input, target):
    batch, num_classes, graph = input.shape
    lanes = graph // _SUB
    x = input.reshape(batch, num_classes, _SUB, lanes)
    t = target.reshape(batch, _SUB, lanes)

    cnt = pl.pallas_call(
        functools.partial(_top2_count_kernel, num_classes=num_classes),
        grid=(batch,),
        in_specs=[
            pl.BlockSpec((1, num_classes, _SUB, lanes), lambda b: (b, 0, 0, 0)),
            pl.BlockSpec((1, _SUB, lanes), lambda b: (b, 0, 0)),
        ],
        out_specs=pl.BlockSpec(
            (1, 1), lambda b: (0, 0), memory_space=pltpu.SMEM
        ),
        out_shape=jax.ShapeDtypeStruct((1, 1), jnp.int32),
    )(x, t)

    edge_acc = cnt[0, 0].astype(jnp.float32) / float(target.size)
    return 1.0 - edge_acc
